# baseline (speedup 1.0000x reference)
import functools

import jax
import jax.numpy as jnp
from jax import lax
from jax.experimental import pallas as pl
from jax.experimental.pallas import tpu as pltpu
from jax.experimental.pallas import tpu_sc as plsc

B = 65536
OBS = 128
H = 256
D = 32
K = 512

BM = 8192


def _enc_block(obs_ref, ew1_ref, eb1_ref, ew2_ref, eb2_ref, ew3_ref, eb3_ref,
               cbm2_ref, c2_ref, iota_ref, idx_ref):
    x = obs_ref[...]
    h = jax.nn.relu(jnp.dot(x, ew1_ref[...], preferred_element_type=jnp.float32)
                    + eb1_ref[...])
    h = jax.nn.relu(jnp.dot(h, ew2_ref[...], preferred_element_type=jnp.float32)
                    + eb2_ref[...])
    z = jnp.dot(h, ew3_ref[...], preferred_element_type=jnp.float32) + eb3_ref[...]
    cross2 = jax.lax.dot_general(z, cbm2_ref[...], (((1,), (1,)), ((), ())),
                                 preferred_element_type=jnp.float32)
    dists = c2_ref[...] + cross2
    m = jnp.min(dists, axis=1, keepdims=True)
    iota = iota_ref[...]
    idx = jnp.min(jnp.where(dists == m, iota, float(K)), axis=1, keepdims=True)
    idx_ref[...] = idx.astype(jnp.int32)


def _dec_block(q_ref, dw1_ref, db1_ref, dw2_ref, db2_ref, dw3_ref, db3_ref,
               out_ref):
    h = jax.nn.relu(jnp.dot(q_ref[...], dw1_ref[...],
                            preferred_element_type=jnp.float32) + db1_ref[...])
    h = jax.nn.relu(jnp.dot(h, dw2_ref[...],
                            preferred_element_type=jnp.float32) + db2_ref[...])
    out_ref[...] = (jnp.dot(h, dw3_ref[...], preferred_element_type=jnp.float32)
                    + db3_ref[...])


def _sc_gather(codebook, idx):
    info = plsc.get_sparse_core_info()
    nc, ns = info.num_cores, info.num_subcores
    nw = nc * ns
    b_per_w = B // nw
    CH = 512
    mesh = plsc.VectorSubcoreMesh(core_axis_name="c", subcore_axis_name="s")

    @functools.partial(
        pl.kernel, mesh=mesh,
        out_type=jax.ShapeDtypeStruct((B, 128), jnp.float32),
        scratch_types=[
            pltpu.VMEM((b_per_w,), jnp.int32),
            pltpu.VMEM((CH, 128), jnp.float32),
            pltpu.SemaphoreType.DMA,
        ],
    )
    def gather_k(table_hbm, idx_hbm, out_hbm, idx_v, rows_v, sem):
        wid = lax.axis_index("s") * nc + lax.axis_index("c")
        base = wid * b_per_w
        pltpu.sync_copy(idx_hbm.at[pl.ds(base, b_per_w)], idx_v)

        @pl.loop(0, b_per_w // CH)
        def body(g):
            off = g * CH
            pltpu.async_copy(table_hbm.at[idx_v.at[pl.ds(off, CH)]],
                             rows_v, sem).wait()
            pltpu.sync_copy(rows_v, out_hbm.at[pl.ds(base + off, CH)])

    return gather_k(codebook, idx)


@jax.jit
def kernel(observations, enc_w1, enc_b1, enc_w2, enc_b2, enc_w3, enc_b3,
           codebook, dec_w1, dec_b1, dec_w2, dec_b2, dec_w3, dec_b3):
    def rep(shape):
        return pl.BlockSpec(shape, lambda i: (0,) * len(shape))

    grid = (B // BM,)
    idx = pl.pallas_call(
        _enc_block,
        grid=grid,
        in_specs=[
            pl.BlockSpec((BM, OBS), lambda i: (i, 0)),
            rep((OBS, H)), rep((1, H)),
            rep((H, H)), rep((1, H)),
            rep((H, D)), rep((1, D)),
            rep((K, D)), rep((1, K)), rep((1, K)),
        ],
        out_specs=pl.BlockSpec((BM, 1), lambda i: (i, 0)),
        out_shape=jax.ShapeDtypeStruct((B, 1), jnp.int32),
    )(observations,
      enc_w1, enc_b1[None, :], enc_w2, enc_b2[None, :], enc_w3, enc_b3[None, :],
      -2.0 * codebook, jnp.sum(codebook * codebook, axis=1)[None, :],
      jnp.arange(K, dtype=jnp.float32)[None, :])

    cb_pad = jnp.zeros((K, 128), jnp.float32).at[:, :D].set(codebook)
    dw1_pad = jnp.zeros((128, H), jnp.float32).at[:D, :].set(dec_w1)
    q = _sc_gather(cb_pad, idx.reshape(B))

    return pl.pallas_call(
        _dec_block,
        grid=grid,
        in_specs=[
            pl.BlockSpec((BM, 128), lambda i: (i, 0)),
            rep((128, H)), rep((1, H)),
            rep((H, H)), rep((1, H)),
            rep((H, OBS)), rep((1, OBS)),
        ],
        out_specs=pl.BlockSpec((BM, OBS), lambda i: (i, 0)),
        out_shape=jax.ShapeDtypeStruct((B, OBS), jnp.float32),
    )(q, dw1_pad, dec_b1[None, :], dec_w2, dec_b2[None, :],
      dec_w3, dec_b3[None, :])


# speedup vs baseline: 7.4068x; 7.4068x over previous
import functools

import jax
import jax.numpy as jnp
from jax.experimental import pallas as pl
from jax.experimental.pallas import tpu as pltpu

B = 65536
OBS = 128
H = 256
D = 32
K = 512

BM = 8192


def _vqvae_block(obs_ref, ew1_ref, eb1_ref, ew2_ref, eb2_ref, ew3_ref, eb3_ref,
                 cb_ref, cbm2_ref, c2_ref, iota_ref, dw1_ref, db1_ref, dw2_ref,
                 db2_ref, dw3_ref, db3_ref, out_ref):
    x = obs_ref[...]
    h = jax.nn.relu(jnp.dot(x, ew1_ref[...], preferred_element_type=jnp.float32)
                    + eb1_ref[...])
    h = jax.nn.relu(jnp.dot(h, ew2_ref[...], preferred_element_type=jnp.float32)
                    + eb2_ref[...])
    z = jnp.dot(h, ew3_ref[...], preferred_element_type=jnp.float32) + eb3_ref[...]

    cross2 = jax.lax.dot_general(z, cbm2_ref[...], (((1,), (1,)), ((), ())),
                                 preferred_element_type=jnp.float32)
    dists = c2_ref[...] + cross2
    m = jnp.min(dists, axis=1, keepdims=True)
    iota = iota_ref[...]
    idx = jnp.min(jnp.where(dists == m, iota, float(K)), axis=1, keepdims=True)
    onehot = jnp.where(iota == idx, 1.0, 0.0)
    q = jnp.dot(onehot, cb_ref[...], preferred_element_type=jnp.float32)

    h = jax.nn.relu(jnp.dot(q, dw1_ref[...], preferred_element_type=jnp.float32)
                    + db1_ref[...])
    h = jax.nn.relu(jnp.dot(h, dw2_ref[...], preferred_element_type=jnp.float32)
                    + db2_ref[...])
    out_ref[...] = (jnp.dot(h, dw3_ref[...], preferred_element_type=jnp.float32)
                    + db3_ref[...])


@jax.jit
def kernel(observations, enc_w1, enc_b1, enc_w2, enc_b2, enc_w3, enc_b3,
           codebook, dec_w1, dec_b1, dec_w2, dec_b2, dec_w3, dec_b3):
    def rep(shape):
        return pl.BlockSpec(shape, lambda i: (0,) * len(shape))

    grid = (B // BM,)
    return pl.pallas_call(
        _vqvae_block,
        grid=grid,
        in_specs=[
            pl.BlockSpec((BM, OBS), lambda i: (i, 0)),
            rep((OBS, H)), rep((1, H)),
            rep((H, H)), rep((1, H)),
            rep((H, D)), rep((1, D)),
            rep((K, D)), rep((K, D)), rep((1, K)), rep((1, K)),
            rep((D, H)), rep((1, H)),
            rep((H, H)), rep((1, H)),
            rep((H, OBS)), rep((1, OBS)),
        ],
        compiler_params=pltpu.CompilerParams(
            dimension_semantics=("parallel",)),
        out_specs=pl.BlockSpec((BM, OBS), lambda i: (i, 0)),
        out_shape=jax.ShapeDtypeStruct((B, OBS), jnp.float32),
    )(observations,
      enc_w1, enc_b1[None, :], enc_w2, enc_b2[None, :], enc_w3, enc_b3[None, :],
      codebook, -2.0 * codebook,
      jnp.sum(codebook * codebook, axis=1)[None, :],
      jnp.arange(K, dtype=jnp.float32)[None, :],
      dec_w1, dec_b1[None, :], dec_w2, dec_b2[None, :], dec_w3, dec_b3[None, :])
